# baseline (speedup 1.0000x reference)
import functools

import jax
import jax.numpy as jnp
from jax import lax
from jax.experimental import pallas as pl
from jax.experimental.pallas import tpu as pltpu
from jax.experimental.pallas import tpu_sc as plsc

TIMESTEPS = 32
MAX_LATENCY = 1.0

_B = 64
_N = 8192
_NC = 2
_NS = 16
_NW = _NC * _NS
_ROWS_PER_TILE = _B // _NS
_BATCH_PER_W = _B // _NW
_CHUNK = 1024
_NCHUNK = _N // _CHUNK
_NSLOT = 3


def _sc_body(data_hbm, out_hbm, d_v, buf, st_v, mm_v, shared_mm, sem):
    c = lax.axis_index("c")
    s = lax.axis_index("s")
    wid = s * _NC + c

    row0 = s * _ROWS_PER_TILE

    def _red_row(carry):
        def _body(j, carry):
            mn, mx = carry
            base = j * 128
            for u in range(8):
                v = d_v[pl.ds(base + u * 16, 16)]
                mn = jnp.minimum(mn, v)
                mx = jnp.maximum(mx, v)
            return mn, mx
        return lax.fori_loop(0, _N // 128, _body, carry)

    pltpu.sync_copy(data_hbm.at[row0], d_v)
    init = (d_v[pl.ds(0, 16)], d_v[pl.ds(0, 16)])
    carry = _red_row(init)
    for r in range(1, _ROWS_PER_TILE):
        pltpu.sync_copy(data_hbm.at[row0 + r], d_v)
        carry = _red_row(carry)
    mn_vec, mx_vec = carry

    st_v[...] = mn_vec
    pltpu.sync_copy(st_v, shared_mm.at[0, s])
    st_v[...] = mx_vec
    pltpu.sync_copy(st_v, shared_mm.at[1, s])
    plsc.subcore_barrier()
    pltpu.sync_copy(shared_mm, mm_v)

    for i in range(_NS):
        mn_vec = jnp.minimum(mn_vec, mm_v[0, i])
        mx_vec = jnp.maximum(mx_vec, mm_v[1, i])
    dmin = jnp.min(mn_vec)
    dmax = jnp.max(mx_vec)
    has_range = dmax > dmin
    denom = jnp.where(has_range, dmax - dmin, jnp.float32(1.0))

    ones = jnp.full((16,), 1.0, jnp.float32)
    zeros = jnp.zeros((16,), jnp.float32)
    col_iota = lax.iota(jnp.int32, 16)

    for sl in range(_NSLOT):
        for t in range(TIMESTEPS):
            @pl.loop(0, _CHUNK // 16, unroll=4)
            def _zero(k, sl=sl, t=t):
                buf[sl, t, pl.ds(k * 16, 16)] = zeros

    def _times(col16):
        x = d_v[pl.ds(col16, 16)]
        normalized = jnp.where(has_range, (x - dmin) / denom,
                               jnp.float32(0.5))
        latencies = (1.0 - normalized) * MAX_LATENCY
        return jnp.clip((latencies * (TIMESTEPS - 1)).astype(jnp.int32),
                        0, TIMESTEPS - 1)

    def _unscatter(sl, ci):
        @pl.loop(0, _CHUNK // 16, unroll=4)
        def _loop(j, sl=sl, ci=ci):
            t = _times(ci * _CHUNK + j * 16)
            plsc.store_scatter(buf.at[sl], [t, j * 16 + col_iota], zeros)

    pending = [None] * _NSLOT
    for bi in range(_BATCH_PER_W):
        b = wid + bi * _NW
        for sl in range(_NSLOT):
            if pending[sl] is not None:
                pending[sl].wait()
                last_ci = max(c2 for c2 in range(_NCHUNK) if c2 % _NSLOT == sl)
                _unscatter(sl, last_ci)
                pending[sl] = None
        pltpu.sync_copy(data_hbm.at[b], d_v)
        for ci in range(_NCHUNK):
            sl = ci % _NSLOT
            if pending[sl] is not None:
                pending[sl].wait()
                _unscatter(sl, ci - _NSLOT)

            @pl.loop(0, _CHUNK // 16, unroll=4)
            def _scatter(j, sl=sl, ci=ci):
                t = _times(ci * _CHUNK + j * 16)
                plsc.store_scatter(buf.at[sl], [t, j * 16 + col_iota], ones)

            pending[sl] = pltpu.async_copy(
                buf.at[sl], out_hbm.at[b, :, pl.ds(ci * _CHUNK, _CHUNK)],
                sem.at[sl])
    for sl in range(_NSLOT):
        if pending[sl] is not None:
            pending[sl].wait()


_sc_latency = pl.kernel(
    _sc_body,
    out_type=jax.ShapeDtypeStruct((_B, TIMESTEPS, _N), jnp.float32),
    mesh=plsc.VectorSubcoreMesh(core_axis_name="c", subcore_axis_name="s"),
    compiler_params=pltpu.CompilerParams(needs_layout_passes=False),
    scratch_types=[
        pltpu.VMEM((_N,), jnp.float32),
        pltpu.VMEM((_NSLOT, TIMESTEPS, _CHUNK), jnp.float32),
        pltpu.VMEM((16,), jnp.float32),
        pltpu.VMEM((2, _NS, 16), jnp.float32),
        pltpu.VMEM_SHARED((2, _NS, 16), jnp.float32),
        pltpu.SemaphoreType.DMA((_NSLOT,)),
    ],
)


def _tc_latency_kernel(data_ref, out_ref, minmax_ref, *, block_n: int):
    i = pl.program_id(0)

    @pl.when(i == 0)
    def _compute_minmax():
        x = data_ref[...]
        minmax_ref[0] = jnp.min(x)
        minmax_ref[1] = jnp.max(x)

    dmin = minmax_ref[0]
    dmax = minmax_ref[1]
    has_range = dmax > dmin
    denom = jnp.where(has_range, dmax - dmin, jnp.float32(1.0))

    x = data_ref[:, pl.ds(i * block_n, block_n)]
    normalized = jnp.where(has_range, (x - dmin) / denom, jnp.float32(0.5))
    latencies = (1.0 - normalized) * MAX_LATENCY
    times = jnp.clip((latencies * (TIMESTEPS - 1)).astype(jnp.int32),
                     0, TIMESTEPS - 1)

    t_iota = jax.lax.broadcasted_iota(
        jnp.int32, (out_ref.shape[0], TIMESTEPS, block_n), 1)
    out_ref[...] = (t_iota == times[:, None, :]).astype(jnp.float32)


def _tc_latency(flat):
    batch, n = flat.shape
    block_n = 512
    while n % block_n:
        block_n //= 2
    grid = n // block_n
    return pl.pallas_call(
        functools.partial(_tc_latency_kernel, block_n=block_n),
        grid=(grid,),
        in_specs=[pl.BlockSpec((batch, n), lambda i: (0, 0))],
        out_specs=pl.BlockSpec((batch, TIMESTEPS, block_n),
                               lambda i: (0, 0, i)),
        out_shape=jax.ShapeDtypeStruct((batch, TIMESTEPS, n), jnp.float32),
        scratch_shapes=[pltpu.SMEM((2,), jnp.float32)],
    )(flat)


def kernel(data):
    squeeze = False
    if data.ndim == 1:
        data = data[None, :]
        squeeze = True
    batch = data.shape[0]
    feat_shape = data.shape[1:]
    flat = data.reshape(batch, -1)
    n = flat.shape[1]

    if (batch, n) == (_B, _N):
        out = _sc_latency(flat)
    else:
        out = _tc_latency(flat)

    out = out.reshape(batch, TIMESTEPS, *feat_shape)
    if squeeze:
        out = out[0]
    return out


# speedup vs baseline: 1.0494x; 1.0494x over previous
import functools

import jax
import jax.numpy as jnp
from jax import lax
from jax.experimental import pallas as pl
from jax.experimental.pallas import tpu as pltpu
from jax.experimental.pallas import tpu_sc as plsc

TIMESTEPS = 32
MAX_LATENCY = 1.0

_B = 64
_N = 8192
_NC = 2
_NS = 16
_NW = _NC * _NS
_ROWS_PER_TILE = _B // _NS
_BATCH_PER_W = _B // _NW
_CHUNK = 1024
_NCHUNK = _N // _CHUNK
_NSLOT = 2


def _sc_body(data_hbm, out_hbm, red_v, d_v, buf, st_v, mm_v,
             shared_mm, sem):
    c = lax.axis_index("c")
    s = lax.axis_index("s")
    wid = s * _NC + c

    row0 = s * _ROWS_PER_TILE
    pltpu.sync_copy(data_hbm.at[pl.ds(row0, _ROWS_PER_TILE)], red_v)

    init = (red_v[0, pl.ds(0, 16)], red_v[0, pl.ds(0, 16)])

    def _red_row(r, carry):
        def _body(j, carry):
            mn, mx = carry
            base = j * 128
            for u in range(8):
                v = red_v[r, pl.ds(base + u * 16, 16)]
                mn = jnp.minimum(mn, v)
                mx = jnp.maximum(mx, v)
            return mn, mx
        return lax.fori_loop(0, _N // 128, _body, carry)

    mn_vec, mx_vec = carry = init
    for r in range(_ROWS_PER_TILE):
        carry = _red_row(r, carry)
    mn_vec, mx_vec = carry

    st_v[...] = mn_vec
    pltpu.sync_copy(st_v, shared_mm.at[0, s])
    st_v[...] = mx_vec
    pltpu.sync_copy(st_v, shared_mm.at[1, s])
    plsc.subcore_barrier()
    pltpu.sync_copy(shared_mm, mm_v)

    for i in range(_NS):
        mn_vec = jnp.minimum(mn_vec, mm_v[0, i])
        mx_vec = jnp.maximum(mx_vec, mm_v[1, i])
    dmin = jnp.min(mn_vec)
    dmax = jnp.max(mx_vec)
    has_range = dmax > dmin
    denom = jnp.where(has_range, dmax - dmin, jnp.float32(1.0))

    ones = jnp.full((16,), 1.0, jnp.float32)
    zeros = jnp.zeros((16,), jnp.float32)
    col_iota = lax.iota(jnp.int32, 16)

    for sl in range(_NSLOT):
        for t in range(TIMESTEPS):
            @pl.loop(0, _CHUNK // 16, unroll=4)
            def _zero(k, sl=sl, t=t):
                buf[sl, t, pl.ds(k * 16, 16)] = zeros

    def _times(col16):
        x = d_v[pl.ds(col16, 16)]
        normalized = jnp.where(has_range, (x - dmin) / denom,
                               jnp.float32(0.5))
        latencies = (1.0 - normalized) * MAX_LATENCY
        return jnp.clip((latencies * (TIMESTEPS - 1)).astype(jnp.int32),
                        0, TIMESTEPS - 1)

    def _unscatter(sl, ci):
        @pl.loop(0, _CHUNK // 16, unroll=4)
        def _loop(j, sl=sl, ci=ci):
            t = _times(ci * _CHUNK + j * 16)
            plsc.store_scatter(buf.at[sl], [t, j * 16 + col_iota], zeros)

    pending = [None] * _NSLOT
    for bi in range(_BATCH_PER_W):
        b = wid + bi * _NW
        for sl in range(_NSLOT):
            if pending[sl] is not None:
                pending[sl].wait()
                last_ci = max(c2 for c2 in range(_NCHUNK) if c2 % _NSLOT == sl)
                _unscatter(sl, last_ci)
                pending[sl] = None
        pltpu.sync_copy(data_hbm.at[b], d_v)
        for ci in range(_NCHUNK):
            sl = ci % _NSLOT
            if pending[sl] is not None:
                pending[sl].wait()
                _unscatter(sl, ci - _NSLOT)

            @pl.loop(0, _CHUNK // 16, unroll=4)
            def _scatter(j, sl=sl, ci=ci):
                t = _times(ci * _CHUNK + j * 16)
                plsc.store_scatter(buf.at[sl], [t, j * 16 + col_iota], ones)

            pending[sl] = pltpu.async_copy(
                buf.at[sl], out_hbm.at[b, :, pl.ds(ci * _CHUNK, _CHUNK)],
                sem.at[sl])
    for sl in range(_NSLOT):
        if pending[sl] is not None:
            pending[sl].wait()


_sc_latency = pl.kernel(
    _sc_body,
    out_type=jax.ShapeDtypeStruct((_B, TIMESTEPS, _N), jnp.float32),
    mesh=plsc.VectorSubcoreMesh(core_axis_name="c", subcore_axis_name="s"),
    compiler_params=pltpu.CompilerParams(needs_layout_passes=False),
    scratch_types=[
        pltpu.VMEM((_ROWS_PER_TILE, _N), jnp.float32),
        pltpu.VMEM((_N,), jnp.float32),
        pltpu.VMEM((_NSLOT, TIMESTEPS, _CHUNK), jnp.float32),
        pltpu.VMEM((16,), jnp.float32),
        pltpu.VMEM((2, _NS, 16), jnp.float32),
        pltpu.VMEM_SHARED((2, _NS, 16), jnp.float32),
        pltpu.SemaphoreType.DMA((_NSLOT,)),
    ],
)


def _tc_latency_kernel(data_ref, out_ref, minmax_ref, *, block_n: int):
    i = pl.program_id(0)

    @pl.when(i == 0)
    def _compute_minmax():
        x = data_ref[...]
        minmax_ref[0] = jnp.min(x)
        minmax_ref[1] = jnp.max(x)

    dmin = minmax_ref[0]
    dmax = minmax_ref[1]
    has_range = dmax > dmin
    denom = jnp.where(has_range, dmax - dmin, jnp.float32(1.0))

    x = data_ref[:, pl.ds(i * block_n, block_n)]
    normalized = jnp.where(has_range, (x - dmin) / denom, jnp.float32(0.5))
    latencies = (1.0 - normalized) * MAX_LATENCY
    times = jnp.clip((latencies * (TIMESTEPS - 1)).astype(jnp.int32),
                     0, TIMESTEPS - 1)

    t_iota = jax.lax.broadcasted_iota(
        jnp.int32, (out_ref.shape[0], TIMESTEPS, block_n), 1)
    out_ref[...] = (t_iota == times[:, None, :]).astype(jnp.float32)


def _tc_latency(flat):
    batch, n = flat.shape
    block_n = 512
    while n % block_n:
        block_n //= 2
    grid = n // block_n
    return pl.pallas_call(
        functools.partial(_tc_latency_kernel, block_n=block_n),
        grid=(grid,),
        in_specs=[pl.BlockSpec((batch, n), lambda i: (0, 0))],
        out_specs=pl.BlockSpec((batch, TIMESTEPS, block_n),
                               lambda i: (0, 0, i)),
        out_shape=jax.ShapeDtypeStruct((batch, TIMESTEPS, n), jnp.float32),
        scratch_shapes=[pltpu.SMEM((2,), jnp.float32)],
    )(flat)


def kernel(data):
    squeeze = False
    if data.ndim == 1:
        data = data[None, :]
        squeeze = True
    batch = data.shape[0]
    feat_shape = data.shape[1:]
    flat = data.reshape(batch, -1)
    n = flat.shape[1]

    if (batch, n) == (_B, _N):
        out = _sc_latency(flat)
    else:
        out = _tc_latency(flat)

    out = out.reshape(batch, TIMESTEPS, *feat_shape)
    if squeeze:
        out = out[0]
    return out
